# baseline (speedup 1.0000x reference)
import functools

import numpy as np

import jax
import jax.numpy as jnp
from jax import lax
from jax.experimental import pallas as pl
from jax.experimental.pallas import tpu as pltpu
from jax.experimental.pallas import tpu_sc as plsc

NC = 2
NS = 16
NW = NC * NS
LANES = 16
CH = 80


def _make_sc_spmm(n, e, d):
    assert e % NW == 0
    epw = e // NW
    assert epw % CH == 0
    nit = epw // CH
    np_ = -(-n // (NS * 8)) * (NS * 8)
    rpt = np_ // NS
    nvec = d // LANES

    mesh = plsc.VectorSubcoreMesh(
        core_axis_name="c", subcore_axis_name="s",
        num_cores=NC, num_subcores=NS)

    @functools.partial(
        pl.kernel,
        out_type=jax.ShapeDtypeStruct((2 * np_, d), jnp.float32),
        mesh=mesh,
        compiler_params=pltpu.CompilerParams(use_tc_tiling_on_sc=False),
        scratch_types=[
            pltpu.VMEM((nit, CH), jnp.int32),
            [pltpu.VMEM((CH,), jnp.int32) for _ in range(2)],
            [pltpu.VMEM((CH,), jnp.float32) for _ in range(2)],
            [pltpu.VMEM((CH, d // 2), jnp.int32) for _ in range(2)],
            [pltpu.VMEM((CH, d), jnp.float32) for _ in range(2)],
            pltpu.VMEM_SHARED((np_, d), jnp.float32),
            [pltpu.SemaphoreType.DMA for _ in range(2)],
            [pltpu.SemaphoreType.DMA for _ in range(2)],
            [pltpu.SemaphoreType.DMA for _ in range(2)],
        ],
    )
    def spmm(feat_hbm, src_hbm, dst_hbm, ew_hbm, out_hbm,
             dsts_v, srcs, ws, rows, frows, acc_sh, isem, gsem, ssem):
        c = lax.axis_index("c")
        s = lax.axis_index("s")
        wid = c * NS + s
        ebase = wid * epw

        zeros = jnp.zeros((LANES,), jnp.float32)

        def zero_row(r, carry):
            for j in range(nvec):
                frows[0][r, pl.ds(j * LANES, LANES)] = zeros
            return carry

        lax.fori_loop(0, CH, zero_row, 0)
        zoff = 0
        while zoff < rpt:
            zn = min(CH, rpt - zoff)
            pltpu.sync_copy(frows[0].at[pl.ds(0, zn)],
                            acc_sh.at[pl.ds(s * rpt + zoff, zn)])
            zoff += zn

        pltpu.sync_copy(dst_hbm.at[wid], dsts_v)
        plsc.subcore_barrier()

        def idx_start(i, b):
            pltpu.make_async_copy(
                src_hbm.at[pl.ds(ebase + i * CH, CH)], srcs[b],
                isem[b]).start()
            pltpu.make_async_copy(
                ew_hbm.at[pl.ds(ebase + i * CH, CH)], ws[b],
                isem[b]).start()

        def idx_wait(i, b):
            pltpu.make_async_copy(
                src_hbm.at[pl.ds(ebase + i * CH, CH)], srcs[b],
                isem[b]).wait()
            pltpu.make_async_copy(
                ew_hbm.at[pl.ds(ebase + i * CH, CH)], ws[b],
                isem[b]).wait()

        def gather_start(i, b):
            pltpu.make_async_copy(
                feat_hbm.at[srcs[b]], rows[b], gsem[b]).start()

        def gather_wait(i, b):
            pltpu.make_async_copy(
                feat_hbm.at[srcs[b]], rows[b], gsem[b]).wait()

        def scatter_start(i, b):
            pltpu.make_async_copy(
                frows[b], acc_sh.at[dsts_v.at[i]], ssem[b]).start(add=True)

        def scatter_wait(i, b):
            pltpu.make_async_copy(
                frows[b], acc_sh.at[dsts_v.at[i]], ssem[b]).wait()

        def scale(i, b):
            for g in range(CH // LANES):
                wvec = ws[b][pl.ds(g * LANES, LANES)]
                e0 = g * LANES
                for l in range(LANES):
                    wl = wvec.at[jnp.full((LANES,), l, jnp.int32)].get(
                        mode="promise_in_bounds")
                    for j in range(d // (2 * LANES)):
                        w16 = rows[b][e0 + l, pl.ds(j * LANES, LANES)]
                        ev = jax.lax.bitcast_convert_type(
                            w16 << 16, jnp.float32)
                        od = jax.lax.bitcast_convert_type(
                            w16 & jnp.int32(-65536), jnp.float32)
                        frows[b][e0 + l, pl.ds(j * 2 * LANES, LANES)] = (
                            ev * wl)
                        frows[b][e0 + l,
                                 pl.ds(j * 2 * LANES + LANES, LANES)] = (
                            od * wl)

        idx_start(0, 0)
        idx_start(1, 1)
        idx_wait(0, 0)
        gather_start(0, 0)

        def step(i, b):
            nb = 1 - b

            @pl.when(i + 1 < nit)
            def _next_gather():
                idx_wait(i + 1, nb)
                gather_start(i + 1, nb)

            gather_wait(i, b)

            @pl.when(i >= 2)
            def _free_frows():
                scatter_wait(i - 2, b)

            scale(i, b)

            @pl.when(i + 2 < nit)
            def _prefetch_idx():
                idx_start(i + 2, b)

            scatter_start(i, b)

        def outer(i0, carry):
            step(2 * i0, 0)
            step(2 * i0 + 1, 1)
            return carry

        lax.fori_loop(0, nit // 2, outer, 0)
        for i in range(2 * (nit // 2), nit):
            step(i, i % 2)
        scatter_wait(nit - 2, (nit - 2) % 2)
        scatter_wait(nit - 1, (nit - 1) % 2)
        plsc.subcore_barrier()

        obase = c * np_ + s * rpt
        woff = 0
        while woff < rpt:
            wn = min(CH, rpt - woff)
            pltpu.sync_copy(acc_sh.at[pl.ds(s * rpt + woff, wn)],
                            out_hbm.at[pl.ds(obase + woff, wn)])
            woff += wn

    return spmm, np_


def _tc_matmul_body(p0_ref, p1_ref, w_ref, b_ref, o_ref):
    acc = p0_ref[...] + p1_ref[...]
    o_ref[...] = (
        jnp.dot(acc, w_ref[...], preferred_element_type=jnp.float32)
        + b_ref[...]
    )


def _make_tc_matmul(np_, d_in, d_out, bm):
    nblk = np_ // bm
    return pl.pallas_call(
        _tc_matmul_body,
        grid=(nblk,),
        in_specs=[
            pl.BlockSpec((bm, d_in), lambda i: (i, 0)),
            pl.BlockSpec((bm, d_in), lambda i, _n=nblk: (i + _n, 0)),
            pl.BlockSpec((d_in, d_out), lambda i: (0, 0)),
            pl.BlockSpec((1, d_out), lambda i: (0, 0)),
        ],
        out_specs=pl.BlockSpec((bm, d_out), lambda i: (i, 0)),
        out_shape=jax.ShapeDtypeStruct((np_, d_out), jnp.float32),
    )


def kernel(features, edge_index, edge_weight, W, bias):
    n, d_in = features.shape
    d_out = W.shape[1]
    e = edge_weight.shape[0]
    epw = e // NW
    nit = epw // CH
    src = edge_index[0].astype(jnp.int32)
    dst = edge_index[1].astype(jnp.int32).reshape(NW, nit, CH)
    ew = edge_weight.astype(jnp.float32)
    feat_i32 = jax.lax.bitcast_convert_type(
        features.astype(jnp.bfloat16).reshape(n, d_in // 2, 2), jnp.int32)
    blk = np.arange(0, 2 * LANES, 2)
    perm = np.concatenate(
        [np.concatenate([2 * LANES * j + blk, 2 * LANES * j + blk + 1])
         for j in range(d_in // (2 * LANES))])
    W_perm = W[perm]

    spmm, np_ = _make_sc_spmm(n, e, d_in)
    partials = spmm(feat_i32, src, dst, ew)
    out_pad = _make_tc_matmul(np_, d_in, d_out, np_ // 8)(
        partials, partials, W_perm, bias.reshape(1, d_out))
    return out_pad[:n]


# speedup vs baseline: 1.0534x; 1.0534x over previous
import functools

import jax
import jax.numpy as jnp
from jax import lax
from jax.experimental import pallas as pl
from jax.experimental.pallas import tpu as pltpu
from jax.experimental.pallas import tpu_sc as plsc

NC = 2
NS = 16
NW = NC * NS
LANES = 16
CH = 80


def _make_sc_spmm(n, e, d):
    assert e % NW == 0
    epw = e // NW
    assert epw % CH == 0
    nit = epw // CH
    np_ = -(-n // (NS * 8)) * (NS * 8)
    rpt = np_ // NS
    nvec = d // LANES

    mesh = plsc.VectorSubcoreMesh(
        core_axis_name="c", subcore_axis_name="s",
        num_cores=NC, num_subcores=NS)

    @functools.partial(
        pl.kernel,
        out_type=jax.ShapeDtypeStruct((2 * np_, d), jnp.float32),
        mesh=mesh,
        scratch_types=[
            pltpu.VMEM((nit, CH), jnp.int32),
            [pltpu.VMEM((CH,), jnp.int32) for _ in range(2)],
            [pltpu.VMEM((CH,), jnp.float32) for _ in range(2)],
            [pltpu.VMEM((CH, d), jnp.float32) for _ in range(2)],
            pltpu.VMEM((CH, d), jnp.float32),
            pltpu.VMEM_SHARED((np_, d), jnp.float32),
            [pltpu.SemaphoreType.DMA for _ in range(2)],
            [pltpu.SemaphoreType.DMA for _ in range(2)],
            [pltpu.SemaphoreType.DMA for _ in range(2)],
        ],
    )
    def spmm(feat_hbm, src_hbm, dst_hbm, ew_hbm, out_hbm,
             dsts_v, srcs, ws, rows, frows, acc_sh, isem, gsem, ssem):
        c = lax.axis_index("c")
        s = lax.axis_index("s")
        wid = c * NS + s
        ebase = wid * epw

        zeros = jnp.zeros((LANES,), jnp.float32)

        def zero_row(r, carry):
            for j in range(nvec):
                frows[r, pl.ds(j * LANES, LANES)] = zeros
            return carry

        lax.fori_loop(0, CH, zero_row, 0)
        zoff = 0
        while zoff < rpt:
            zn = min(CH, rpt - zoff)
            pltpu.sync_copy(frows.at[pl.ds(0, zn)],
                            acc_sh.at[pl.ds(s * rpt + zoff, zn)])
            zoff += zn

        pltpu.sync_copy(dst_hbm.at[wid], dsts_v)
        plsc.subcore_barrier()

        def idx_start(i, b):
            pltpu.make_async_copy(
                src_hbm.at[pl.ds(ebase + i * CH, CH)], srcs[b],
                isem[b]).start()
            pltpu.make_async_copy(
                ew_hbm.at[pl.ds(ebase + i * CH, CH)], ws[b],
                isem[b]).start()

        def idx_wait(i, b):
            pltpu.make_async_copy(
                src_hbm.at[pl.ds(ebase + i * CH, CH)], srcs[b],
                isem[b]).wait()
            pltpu.make_async_copy(
                ew_hbm.at[pl.ds(ebase + i * CH, CH)], ws[b],
                isem[b]).wait()

        def gather_start(i, b):
            pltpu.make_async_copy(
                feat_hbm.at[srcs[b]], rows[b], gsem[b]).start()

        def gather_wait(i, b):
            pltpu.make_async_copy(
                feat_hbm.at[srcs[b]], rows[b], gsem[b]).wait()

        def scatter_start(i, b):
            pltpu.make_async_copy(
                rows[b], acc_sh.at[dsts_v.at[i]], ssem[b]).start(add=True)

        def scatter_wait(i, b):
            pltpu.make_async_copy(
                rows[b], acc_sh.at[dsts_v.at[i]], ssem[b]).wait()

        def scale(i, b):
            for g in range(CH // LANES):
                wvec = ws[b][pl.ds(g * LANES, LANES)]
                e0 = g * LANES
                for l in range(LANES):
                    wl = wvec.at[jnp.full((LANES,), l, jnp.int32)].get(
                        mode="promise_in_bounds")
                    for j in range(nvec):
                        sl = pl.ds(j * LANES, LANES)
                        rows[b][e0 + l, sl] = rows[b][e0 + l, sl] * wl

        idx_start(0, 0)
        idx_start(1, 1)
        idx_wait(0, 0)
        gather_start(0, 0)

        def step(i, b):
            nb = 1 - b

            @pl.when(jnp.logical_and(i >= 1, i + 1 < nit))
            def _free_rows():
                scatter_wait(i - 1, nb)

            @pl.when(i + 1 < nit)
            def _next_gather():
                idx_wait(i + 1, nb)
                gather_start(i + 1, nb)

            gather_wait(i, b)
            scale(i, b)

            @pl.when(i + 2 < nit)
            def _prefetch_idx():
                idx_start(i + 2, b)

            scatter_start(i, b)

        def outer(i0, carry):
            step(2 * i0, 0)
            step(2 * i0 + 1, 1)
            return carry

        lax.fori_loop(0, nit // 2, outer, 0)
        for i in range(2 * (nit // 2), nit):
            step(i, i % 2)
        scatter_wait(nit - 2, (nit - 2) % 2)
        scatter_wait(nit - 1, (nit - 1) % 2)
        plsc.subcore_barrier()

        obase = c * np_ + s * rpt
        woff = 0
        while woff < rpt:
            wn = min(CH, rpt - woff)
            pltpu.sync_copy(acc_sh.at[pl.ds(s * rpt + woff, wn)],
                            out_hbm.at[pl.ds(obase + woff, wn)])
            woff += wn

    return spmm, np_


def _tc_matmul_body(p0_ref, p1_ref, w_ref, b_ref, o_ref):
    acc = p0_ref[...] + p1_ref[...]
    o_ref[...] = (
        jnp.dot(acc, w_ref[...], preferred_element_type=jnp.float32)
        + b_ref[...]
    )


def _make_tc_matmul(np_, d_in, d_out, bm):
    nblk = np_ // bm
    return pl.pallas_call(
        _tc_matmul_body,
        grid=(nblk,),
        in_specs=[
            pl.BlockSpec((bm, d_in), lambda i: (i, 0)),
            pl.BlockSpec((bm, d_in), lambda i, _n=nblk: (i + _n, 0)),
            pl.BlockSpec((d_in, d_out), lambda i: (0, 0)),
            pl.BlockSpec((1, d_out), lambda i: (0, 0)),
        ],
        out_specs=pl.BlockSpec((bm, d_out), lambda i: (i, 0)),
        out_shape=jax.ShapeDtypeStruct((np_, d_out), jnp.float32),
    )


def kernel(features, edge_index, edge_weight, W, bias):
    n, d_in = features.shape
    d_out = W.shape[1]
    e = edge_weight.shape[0]
    epw = e // NW
    nit = epw // CH
    src = edge_index[0].astype(jnp.int32)
    dst = edge_index[1].astype(jnp.int32).reshape(NW, nit, CH)
    ew = edge_weight.astype(jnp.float32)

    spmm, np_ = _make_sc_spmm(n, e, d_in)
    partials = spmm(features, src, dst, ew)
    out_pad = _make_tc_matmul(np_, d_in, d_out, np_ // 8)(
        partials, partials, W, bias.reshape(1, d_out))
    return out_pad[:n]
